# baseline (speedup 1.0000x reference)
import jax
import jax.numpy as jnp
from jax.experimental import pallas as pl
from jax.experimental.pallas import tpu as pltpu

N = 10000
B = 16
NPG = 625
NPAD = 640
K = 16
EPS = 1e-5
NEDGE = float(N * K)
BIG = 1e30

_f32 = jnp.float32
_HI = jax.lax.Precision.HIGHEST


def _bn_apply(h, s_all, q_all, g, be):
    m = jnp.sum(s_all, axis=0) / NEDGE
    q = jnp.sum(q_all, axis=0) / NEDGE
    v = q - m * m
    return (h - m) / jnp.sqrt(v + EPS) * g + be


def _row_mask():
    return (jax.lax.broadcasted_iota(jnp.int32, (NPAD, 1), 0) < NPG).astype(_f32)


def _bn0_kernel(x_ref, g_ref, b_ref, o_ref):
    x = x_ref[...]
    m = jnp.mean(x, axis=0, keepdims=True)
    xc = x - m
    v = jnp.mean(xc * xc, axis=0, keepdims=True)
    o_ref[...] = xc / jnp.sqrt(v + EPS) * g_ref[...] + b_ref[...]


def _edge_mask():
    rm = jax.lax.broadcasted_iota(jnp.int32, (K, NPAD, 1), 1) < NPG
    return jnp.reshape(rm.astype(_f32), (K * NPAD, 1))


def _knn_l1_kernel(f_ref, w1_ref, b1_ref, h_ref, s_ref, q_ref, oh_ref):
    f = f_ref[0]
    sq = jnp.sum(f * f, axis=1, keepdims=True)
    fft = jax.lax.dot_general(f, f, (((1,), (1,)), ((), ())),
                              preferred_element_type=_f32)
    d2 = sq + jnp.transpose(sq) - 2.0 * fft
    row = jax.lax.broadcasted_iota(jnp.int32, (NPAD, NPAD), 0)
    d2 = jnp.where(row >= NPG, BIG, d2)

    for k in range(K):
        m = jnp.min(d2, axis=0, keepdims=True)
        idx = jnp.min(jnp.where(d2 == m, row, NPAD), axis=0, keepdims=True)
        oh = (row == idx)
        d2 = jnp.where(oh, BIG, d2)
        oh_ref[k] = oh.astype(jnp.bfloat16)

    bf = jnp.bfloat16
    f_hi = f.astype(bf)
    r1 = f - f_hi.astype(_f32)
    f_mid = r1.astype(bf)
    f_lo = (r1 - f_mid.astype(_f32)).astype(bf)
    ohm = oh_ref[...]
    dn = (((1,), (0,)), ((), ()))
    xj = ((jax.lax.dot_general(ohm, f_hi, dn, preferred_element_type=_f32)
           + jax.lax.dot_general(ohm, f_mid, dn, preferred_element_type=_f32))
          + jax.lax.dot_general(ohm, f_lo, dn, preferred_element_type=_f32))
    xj = jnp.reshape(xj, (K * NPAD, f.shape[1]))
    ft = jnp.reshape(jnp.broadcast_to(f[None], (K, NPAD, f.shape[1])),
                     (K * NPAD, f.shape[1]))
    e = jnp.concatenate([ft, xj - ft], axis=1)
    h = jnp.dot(e, w1_ref[...], preferred_element_type=_f32) + b1_ref[...]
    dh = h.shape[1]
    h_ref[0] = jnp.reshape(h, (K, NPAD, dh))
    hm = h * _edge_mask()
    s_ref[0] = jnp.sum(hm, axis=0, keepdims=True)
    q_ref[0] = jnp.sum(h * hm, axis=0, keepdims=True)


def _mid_kernel(h_ref, s_ref, q_ref, g_ref, be_ref, w_ref, b_ref,
                ho_ref, so_ref, qo_ref):
    s_all = s_ref[...][:, 0, :]
    q_all = q_ref[...][:, 0, :]
    g = g_ref[...]
    be = be_ref[...]
    w = w_ref[...]
    b = b_ref[...]
    dhi = w.shape[0]
    dho = w.shape[1]
    h = jnp.reshape(h_ref[0], (K * NPAD, dhi))
    hn = jnp.maximum(_bn_apply(h, s_all, q_all, g, be), 0.0)
    ho = jnp.dot(hn, w, preferred_element_type=_f32) + b
    ho_ref[0] = jnp.reshape(ho, (K, NPAD, dho))
    hm = ho * _edge_mask()
    so_ref[0] = jnp.sum(hm, axis=0, keepdims=True)
    qo_ref[0] = jnp.sum(ho * hm, axis=0, keepdims=True)


def _fin_kernel(h_ref, s_ref, q_ref, g_ref, be_ref, x2_ref):
    s_all = s_ref[...][:, 0, :]
    q_all = q_ref[...][:, 0, :]
    g = g_ref[...]
    be = be_ref[...]
    dh = g.shape[1]
    acc = jnp.zeros((NPAD, dh), _f32)
    for k in range(K):
        acc = acc + jnp.maximum(_bn_apply(h_ref[0, k], s_all, q_all, g, be), 0.0)
    x2_ref[0] = acc * (1.0 / K) * _row_mask()


def _fin3_kernel(h_ref, s_ref, q_ref, g_ref, be_ref, xs_ref):
    s_all = s_ref[...][:, 0, :]
    q_all = q_ref[...][:, 0, :]
    g = g_ref[...]
    be = be_ref[...]
    dh = g.shape[1]
    acc = jnp.zeros((NPAD, dh), _f32)
    for k in range(K):
        acc = acc + jnp.maximum(_bn_apply(h_ref[0, k], s_all, q_all, g, be), 0.0)
    x2 = acc * (1.0 / K) * _row_mask()
    xs_ref[0] = jnp.sum(x2, axis=0, keepdims=True)


def _readout_kernel(x1g_ref, xs_ref, u_ref, gg_ref, gb_ref,
                    w1_ref, b1_ref, g1_ref, be1_ref, w2_ref, b2_ref, o_ref):
    u = u_ref[...]
    mu = jnp.mean(u, axis=0, keepdims=True)
    uc = u - mu
    vu = jnp.mean(uc * uc, axis=0, keepdims=True)
    u1 = uc / jnp.sqrt(vu + EPS) * gg_ref[...] + gb_ref[...]

    x1g = x1g_ref[...]
    rm = (jax.lax.broadcasted_iota(jnp.int32, (1, NPAD, 1), 1) < NPG)
    x1m = jnp.sum(x1g * rm.astype(_f32), axis=1) / float(NPG)
    x2m = jnp.reshape(xs_ref[...], (B, -1)) / float(NPG)

    ucat = jnp.concatenate([u1, x1m, x2m], axis=1)
    h = jnp.dot(ucat, w1_ref[...], preferred_element_type=_f32) + b1_ref[...]
    mh = jnp.mean(h, axis=0, keepdims=True)
    hc = h - mh
    vh = jnp.mean(hc * hc, axis=0, keepdims=True)
    hn = jnp.maximum(hc / jnp.sqrt(vh + EPS) * g1_ref[...] + be1_ref[...], 0.0)
    o_ref[...] = jnp.dot(hn, w2_ref[...], preferred_element_type=_f32) + b2_ref[...]


def _full(shape):
    return pl.BlockSpec(shape, lambda *_: tuple(0 for _ in shape))


def _per_graph(shape):
    return pl.BlockSpec(shape, lambda b: (b,) + tuple(0 for _ in shape[1:]))


_PARAMS = pltpu.CompilerParams(dimension_semantics=("parallel",))


def _conv_layer1(f, w1, b1, dh):
    din = f.shape[-1]
    return pl.pallas_call(
        _knn_l1_kernel,
        grid=(B,),
        in_specs=[_per_graph((1, NPAD, din)), _full(w1.shape), _full((1, dh))],
        out_specs=[_per_graph((1, K, NPAD, dh)), _per_graph((1, 1, dh)),
                   _per_graph((1, 1, dh))],
        out_shape=[jax.ShapeDtypeStruct((B, K, NPAD, dh), _f32),
                   jax.ShapeDtypeStruct((B, 1, dh), _f32),
                   jax.ShapeDtypeStruct((B, 1, dh), _f32)],
        scratch_shapes=[pltpu.VMEM((K, NPAD, NPAD), jnp.bfloat16)],
        compiler_params=_PARAMS,
    )(f, w1, b1)


def _conv_mid(h, s, q, g, be, w, b):
    dhi = h.shape[-1]
    dho = w.shape[1]
    return pl.pallas_call(
        _mid_kernel,
        grid=(B,),
        in_specs=[_per_graph((1, K, NPAD, dhi)), _full((B, 1, dhi)),
                  _full((B, 1, dhi)), _full((1, dhi)), _full((1, dhi)),
                  _full((dhi, dho)), _full((1, dho))],
        out_specs=[_per_graph((1, K, NPAD, dho)), _per_graph((1, 1, dho)),
                   _per_graph((1, 1, dho))],
        out_shape=[jax.ShapeDtypeStruct((B, K, NPAD, dho), _f32),
                   jax.ShapeDtypeStruct((B, 1, dho), _f32),
                   jax.ShapeDtypeStruct((B, 1, dho), _f32)],
        compiler_params=_PARAMS,
    )(h, s, q, g, be, w, b)


def _conv_fin(h, s, q, g, be, last):
    dh = h.shape[-1]
    body = _fin3_kernel if last else _fin_kernel
    out_spec = _per_graph((1, 1, dh)) if last else _per_graph((1, NPAD, dh))
    out_shape = (jax.ShapeDtypeStruct((B, 1, dh), _f32) if last
                 else jax.ShapeDtypeStruct((B, NPAD, dh), _f32))
    return pl.pallas_call(
        body,
        grid=(B,),
        in_specs=[_per_graph((1, K, NPAD, dh)), _full((B, 1, dh)),
                  _full((B, 1, dh)), _full((1, dh)), _full((1, dh))],
        out_specs=out_spec,
        out_shape=out_shape,
        compiler_params=_PARAMS,
    )(h, s, q, g, be)


def _run_conv(f, p, dh, last):
    r = lambda a: jnp.reshape(a, (1, -1))
    h1, s1, q1 = _conv_layer1(f, p['W1'], r(p['b1']), dh)
    h2, s2, q2 = _conv_mid(h1, s1, q1, r(p['g1']), r(p['be1']), p['W2'], r(p['b2']))
    h3, s3, q3 = _conv_mid(h2, s2, q2, r(p['g2']), r(p['be2']), p['W3'], r(p['b3']))
    return _conv_fin(h3, s3, q3, r(p['g3']), r(p['be3']), last)


def kernel(x, u, batch, params):
    del batch
    r = lambda a: jnp.reshape(a, (1, -1))
    pb = params['bn0']
    x1 = pl.pallas_call(
        _bn0_kernel,
        in_specs=[_full((N, 3)), _full((1, 3)), _full((1, 3))],
        out_specs=_full((N, 3)),
        out_shape=jax.ShapeDtypeStruct((N, 3), _f32),
    )(x, r(pb['g']), r(pb['b']))

    pad = lambda a: jnp.pad(jnp.reshape(a, (B, NPG, a.shape[-1])),
                            ((0, 0), (0, NPAD - NPG), (0, 0)))
    xg = pad(x)
    x1g = pad(x1)

    x2g = _run_conv(xg, params['conv1'], 32, last=False)
    x2g = _run_conv(jnp.concatenate([x1g, x2g], axis=-1), params['conv2'], 64,
                    last=False)
    xs = _run_conv(jnp.concatenate([x1g, x2g], axis=-1), params['conv3'], 128,
                   last=True)

    po = params['out']
    pg = params['bng']
    return pl.pallas_call(
        _readout_kernel,
        in_specs=[_full((B, NPAD, 3)), _full((B, 1, 128)), _full((B, 2)),
                  _full((1, 2)), _full((1, 2)), _full(po['W1'].shape),
                  _full((1, 256)), _full((1, 256)), _full((1, 256)),
                  _full(po['W2'].shape), _full((1, 1))],
        out_specs=_full((B, 1)),
        out_shape=jax.ShapeDtypeStruct((B, 1), _f32),
    )(x1g, xs, u, r(pg['g']), r(pg['b']), po['W1'], r(po['b1']),
      r(po['g1']), r(po['be1']), po['W2'], r(po['b2']))


# speedup vs baseline: 1.1876x; 1.1876x over previous
import jax
import jax.numpy as jnp
import numpy as np
from jax.experimental import pallas as pl
from jax.experimental.pallas import tpu as pltpu

N = 10000
B = 16
NPG = 625
NPAD = 640
K = 16
EPS = 1e-5
NEDGE = float(N * K)
BIG = 1e30

_f32 = jnp.float32
_HI = jax.lax.Precision.HIGHEST


def _bn_apply(h, s_all, q_all, g, be):
    m = jnp.sum(s_all, axis=0) / NEDGE
    q = jnp.sum(q_all, axis=0) / NEDGE
    v = q - m * m
    return (h - m) / jnp.sqrt(v + EPS) * g + be


def _row_mask():
    return (jax.lax.broadcasted_iota(jnp.int32, (NPAD, 1), 0) < NPG).astype(_f32)


def _bn0_kernel(x_ref, g_ref, b_ref, o_ref):
    x = x_ref[...]
    m = jnp.mean(x, axis=0, keepdims=True)
    xc = x - m
    v = jnp.mean(xc * xc, axis=0, keepdims=True)
    o_ref[...] = xc / jnp.sqrt(v + EPS) * g_ref[...] + b_ref[...]


def _edge_mask():
    rm = jax.lax.broadcasted_iota(jnp.int32, (K, NPAD, 1), 1) < NPG
    return jnp.reshape(rm.astype(_f32), (K * NPAD, 1))


def _knn_l1_kernel(f_ref, w1_ref, b1_ref, h_ref, s_ref, q_ref, oh_ref):
    f = f_ref[0]
    sq = jnp.sum(f * f, axis=1, keepdims=True)
    fft = jax.lax.dot_general(f, f, (((1,), (1,)), ((), ())),
                              preferred_element_type=_f32)
    d2 = sq + jnp.transpose(sq) - 2.0 * fft
    col = jax.lax.broadcasted_iota(jnp.int32, (NPAD, NPAD), 1)
    d2 = jnp.where(col >= NPG, BIG, d2)

    for k in range(K):
        m = jnp.min(d2, axis=1, keepdims=True)
        idx = jnp.min(jnp.where(d2 == m, col, NPAD), axis=1, keepdims=True)
        oh = (col == idx)
        d2 = jnp.where(oh, BIG, d2)
        oh_ref[k] = oh.astype(jnp.bfloat16)

    bf = jnp.bfloat16
    f_hi = f.astype(bf)
    r1 = f - f_hi.astype(_f32)
    f_mid = r1.astype(bf)
    f_lo = (r1 - f_mid.astype(_f32)).astype(bf)
    ohm = jnp.reshape(oh_ref[...], (K * NPAD, NPAD))
    xj = ((jnp.dot(ohm, f_hi, preferred_element_type=_f32)
           + jnp.dot(ohm, f_mid, preferred_element_type=_f32))
          + jnp.dot(ohm, f_lo, preferred_element_type=_f32))
    ft = jnp.reshape(jnp.broadcast_to(f[None], (K, NPAD, f.shape[1])),
                     (K * NPAD, f.shape[1]))
    e = jnp.concatenate([ft, xj - ft], axis=1)
    h = jnp.dot(e, w1_ref[...], preferred_element_type=_f32) + b1_ref[...]
    dh = h.shape[1]
    h_ref[0] = jnp.reshape(h, (K, NPAD, dh))
    hm = h * _edge_mask()
    s_ref[0] = jnp.sum(hm, axis=0, keepdims=True)
    q_ref[0] = jnp.sum(h * hm, axis=0, keepdims=True)


def _mid_kernel(h_ref, s_ref, q_ref, g_ref, be_ref, w_ref, b_ref,
                ho_ref, so_ref, qo_ref):
    s_all = s_ref[...][:, 0, :]
    q_all = q_ref[...][:, 0, :]
    g = g_ref[...]
    be = be_ref[...]
    w = w_ref[...]
    b = b_ref[...]
    dhi = w.shape[0]
    dho = w.shape[1]
    h = jnp.reshape(h_ref[0], (K * NPAD, dhi))
    hn = jnp.maximum(_bn_apply(h, s_all, q_all, g, be), 0.0)
    ho = jnp.dot(hn, w, preferred_element_type=_f32) + b
    ho_ref[0] = jnp.reshape(ho, (K, NPAD, dho))
    hm = ho * _edge_mask()
    so_ref[0] = jnp.sum(hm, axis=0, keepdims=True)
    qo_ref[0] = jnp.sum(ho * hm, axis=0, keepdims=True)


def _fin_kernel(h_ref, s_ref, q_ref, g_ref, be_ref, x2_ref):
    s_all = s_ref[...][:, 0, :]
    q_all = q_ref[...][:, 0, :]
    g = g_ref[...]
    be = be_ref[...]
    dh = g.shape[1]
    acc = jnp.zeros((NPAD, dh), _f32)
    for k in range(K):
        acc = acc + jnp.maximum(_bn_apply(h_ref[0, k], s_all, q_all, g, be), 0.0)
    x2_ref[0] = acc * (1.0 / K) * _row_mask()


def _fin3_kernel(h_ref, s_ref, q_ref, g_ref, be_ref, xs_ref):
    s_all = s_ref[...][:, 0, :]
    q_all = q_ref[...][:, 0, :]
    g = g_ref[...]
    be = be_ref[...]
    dh = g.shape[1]
    acc = jnp.zeros((NPAD, dh), _f32)
    for k in range(K):
        acc = acc + jnp.maximum(_bn_apply(h_ref[0, k], s_all, q_all, g, be), 0.0)
    x2 = acc * (1.0 / K) * _row_mask()
    xs_ref[0] = jnp.sum(x2, axis=0, keepdims=True)


def _readout_kernel(x1g_ref, xs_ref, u_ref, gg_ref, gb_ref,
                    w1_ref, b1_ref, g1_ref, be1_ref, w2_ref, b2_ref, o_ref):
    u = u_ref[...]
    mu = jnp.mean(u, axis=0, keepdims=True)
    uc = u - mu
    vu = jnp.mean(uc * uc, axis=0, keepdims=True)
    u1 = uc / jnp.sqrt(vu + EPS) * gg_ref[...] + gb_ref[...]

    x1g = x1g_ref[...]
    rm = (jax.lax.broadcasted_iota(jnp.int32, (1, NPAD, 1), 1) < NPG)
    x1m = jnp.sum(x1g * rm.astype(_f32), axis=1) / float(NPG)
    x2m = jnp.reshape(xs_ref[...], (B, -1)) / float(NPG)

    ucat = jnp.concatenate([u1, x1m, x2m], axis=1)
    h = jnp.dot(ucat, w1_ref[...], preferred_element_type=_f32) + b1_ref[...]
    mh = jnp.mean(h, axis=0, keepdims=True)
    hc = h - mh
    vh = jnp.mean(hc * hc, axis=0, keepdims=True)
    hn = jnp.maximum(hc / jnp.sqrt(vh + EPS) * g1_ref[...] + be1_ref[...], 0.0)
    o_ref[...] = jnp.dot(hn, w2_ref[...], preferred_element_type=_f32) + b2_ref[...]


def _full(shape):
    return pl.BlockSpec(shape, lambda *_: tuple(0 for _ in shape))


def _per_graph(shape):
    return pl.BlockSpec(shape, lambda b: (b,) + tuple(0 for _ in shape[1:]))


_PARAMS = pltpu.CompilerParams(dimension_semantics=("parallel",))


def _conv_layer1(f, w1, b1, dh):
    din = f.shape[-1]
    nb = f.shape[0]
    return pl.pallas_call(
        _knn_l1_kernel,
        grid=(nb,),
        in_specs=[_per_graph((1, NPAD, din)), _full(w1.shape), _full((1, dh))],
        out_specs=[_per_graph((1, K, NPAD, dh)), _per_graph((1, 1, dh)),
                   _per_graph((1, 1, dh))],
        out_shape=[jax.ShapeDtypeStruct((nb, K, NPAD, dh), _f32),
                   jax.ShapeDtypeStruct((nb, 1, dh), _f32),
                   jax.ShapeDtypeStruct((nb, 1, dh), _f32)],
        scratch_shapes=[pltpu.VMEM((K, NPAD, NPAD), jnp.bfloat16)],
        compiler_params=_PARAMS,
    )(f, w1, b1)


def _conv_mid(h, s, q, g, be, w, b):
    dhi = h.shape[-1]
    dho = w.shape[1]
    nb = h.shape[0]
    return pl.pallas_call(
        _mid_kernel,
        grid=(nb,),
        in_specs=[_per_graph((1, K, NPAD, dhi)), _full((1, 1, dhi)),
                  _full((1, 1, dhi)), _full((1, dhi)), _full((1, dhi)),
                  _full((dhi, dho)), _full((1, dho))],
        out_specs=[_per_graph((1, K, NPAD, dho)), _per_graph((1, 1, dho)),
                   _per_graph((1, 1, dho))],
        out_shape=[jax.ShapeDtypeStruct((nb, K, NPAD, dho), _f32),
                   jax.ShapeDtypeStruct((nb, 1, dho), _f32),
                   jax.ShapeDtypeStruct((nb, 1, dho), _f32)],
        compiler_params=_PARAMS,
    )(h, s, q, g, be, w, b)


def _conv_fin(h, s, q, g, be, last):
    dh = h.shape[-1]
    nb = h.shape[0]
    body = _fin3_kernel if last else _fin_kernel
    out_spec = _per_graph((1, 1, dh)) if last else _per_graph((1, NPAD, dh))
    out_shape = (jax.ShapeDtypeStruct((nb, 1, dh), _f32) if last
                 else jax.ShapeDtypeStruct((nb, NPAD, dh), _f32))
    return pl.pallas_call(
        body,
        grid=(nb,),
        in_specs=[_per_graph((1, K, NPAD, dh)), _full((1, 1, dh)),
                  _full((1, 1, dh)), _full((1, dh)), _full((1, dh))],
        out_specs=out_spec,
        out_shape=out_shape,
        compiler_params=_PARAMS,
    )(h, s, q, g, be)


def _tot(a):
    return jax.lax.psum(jnp.sum(a, axis=0, keepdims=True), 'd')


def _run_conv(f, p, dh, last):
    r = lambda a: jnp.reshape(a, (1, -1))
    h1, s1, q1 = _conv_layer1(f, p['W1'], r(p['b1']), dh)
    h2, s2, q2 = _conv_mid(h1, _tot(s1), _tot(q1), r(p['g1']), r(p['be1']),
                           p['W2'], r(p['b2']))
    h3, s3, q3 = _conv_mid(h2, _tot(s2), _tot(q2), r(p['g2']), r(p['be2']),
                           p['W3'], r(p['b3']))
    return _conv_fin(h3, _tot(s3), _tot(q3), r(p['g3']), r(p['be3']), last)


def _forward_local(x, u, params):
    r = lambda a: jnp.reshape(a, (1, -1))
    pb = params['bn0']
    x1 = pl.pallas_call(
        _bn0_kernel,
        in_specs=[_full((N, 3)), _full((1, 3)), _full((1, 3))],
        out_specs=_full((N, 3)),
        out_shape=jax.ShapeDtypeStruct((N, 3), _f32),
    )(x, r(pb['g']), r(pb['b']))

    pad = lambda a: jnp.pad(jnp.reshape(a, (B, NPG, a.shape[-1])),
                            ((0, 0), (0, NPAD - NPG), (0, 0)))
    xg_all = pad(x)
    x1g_all = pad(x1)
    nd = jax.lax.axis_size('d')
    nb = B // nd
    di = jax.lax.axis_index('d')
    xg = jax.lax.dynamic_slice_in_dim(xg_all, di * nb, nb, 0)
    x1g = jax.lax.dynamic_slice_in_dim(x1g_all, di * nb, nb, 0)

    x2g = _run_conv(xg, params['conv1'], 32, last=False)
    x2g = _run_conv(jnp.concatenate([x1g, x2g], axis=-1), params['conv2'], 64,
                    last=False)
    xs = _run_conv(jnp.concatenate([x1g, x2g], axis=-1), params['conv3'], 128,
                   last=True)
    xs = jax.lax.all_gather(xs, 'd', axis=0, tiled=True)

    po = params['out']
    pg = params['bng']
    return pl.pallas_call(
        _readout_kernel,
        in_specs=[_full((B, NPAD, 3)), _full((B, 1, 128)), _full((B, 2)),
                  _full((1, 2)), _full((1, 2)), _full(po['W1'].shape),
                  _full((1, 256)), _full((1, 256)), _full((1, 256)),
                  _full(po['W2'].shape), _full((1, 1))],
        out_specs=_full((B, 1)),
        out_shape=jax.ShapeDtypeStruct((B, 1), _f32),
    )(x1g_all, xs, u, r(pg['g']), r(pg['b']), po['W1'], r(po['b1']),
      r(po['g1']), r(po['be1']), po['W2'], r(po['b2']))


def kernel(x, u, batch, params):
    del batch
    devs = jax.devices()
    ndev = 2 if len(devs) >= 2 else 1
    mesh = jax.sharding.Mesh(np.array(devs[:ndev]), ('d',))
    P = jax.sharding.PartitionSpec
    fn = jax.shard_map(_forward_local, mesh=mesh,
                       in_specs=(P(), P(), P()), out_specs=P(),
                       check_vma=False)
    return fn(x, u, params)
